# baseline (speedup 1.0000x reference)
import jax
import jax.numpy as jnp
from jax.experimental import pallas as pl
from jax.experimental.pallas import tpu as pltpu

_TILE = 1024


def _gate_kernel(x_ref, gw_ref, idx_ref, rw_ref):
    x = x_ref[...]
    logits = jax.lax.dot_general(
        x, gw_ref[...], (((1,), (1,)), ((), ())),
        preferred_element_type=jnp.float32)
    s = jnp.sum(logits, axis=0, keepdims=True)
    e_iota = jax.lax.broadcasted_iota(jnp.int32, s.shape, 1)
    i0 = jnp.argmax(s, axis=1)[0]
    s_masked = jnp.where(e_iota == i0, -jnp.inf, s)
    i1 = jnp.argmax(s_masked, axis=1)[0]

    l0 = jnp.sum(jnp.where(e_iota == i0, logits, 0.0), axis=1, keepdims=True)
    l1 = jnp.sum(jnp.where(e_iota == i1, logits, 0.0), axis=1, keepdims=True)
    m = jnp.maximum(l0, l1)
    e0 = jnp.exp(l0 - m)
    e1 = jnp.exp(l1 - m)
    denom = e0 + e1
    w0 = e0 / denom
    w1 = e1 / denom

    k_iota = jax.lax.broadcasted_iota(jnp.int32, (1, 2), 1)
    idx_ref[...] = jnp.where(k_iota == 0, i0, i1).astype(jnp.int32)
    rw_ref[...] = jnp.concatenate([w0, w1], axis=1)


def _expert_kernel(idx_ref, x_ref, w0_ref, w1_ref, b0_ref, b1_ref, rw_ref,
                   out_ref):
    x = x_ref[...]
    part0 = jax.lax.dot_general(
        x, w0_ref[0], (((1,), (1,)), ((), ())),
        preferred_element_type=jnp.float32)
    part1 = jax.lax.dot_general(
        x, w1_ref[0], (((1,), (1,)), ((), ())),
        preferred_element_type=jnp.float32)
    rw = rw_ref[...]
    w0 = rw[:, 0:1]
    w1 = rw[:, 1:2]
    j = pl.program_id(0)
    b0 = b0_ref[0, j, :].reshape(1, -1)
    b1 = b1_ref[0, j, :].reshape(1, -1)
    out_ref[...] = w0 * (part0 + b0) + w1 * (part1 + b1)


@jax.jit
def kernel(hidden_states, gate_w, expert_w, expert_b):
    B, S, d = hidden_states.shape
    T = B * S
    x = hidden_states.reshape(T, d)

    idx2d, rw = pl.pallas_call(
        _gate_kernel,
        out_shape=(
            jax.ShapeDtypeStruct((1, 2), jnp.int32),
            jax.ShapeDtypeStruct((T, 2), jnp.float32),
        ),
    )(x, gate_w)
    idx = idx2d.reshape(2)

    n_tiles = d // _TILE
    b3 = expert_b.reshape(expert_b.shape[0], n_tiles, _TILE)
    out = pl.pallas_call(
        _expert_kernel,
        grid_spec=pltpu.PrefetchScalarGridSpec(
            num_scalar_prefetch=1,
            grid=(n_tiles,),
            in_specs=[
                pl.BlockSpec((T, d), lambda j, idx_ref: (0, 0)),
                pl.BlockSpec((1, _TILE, d),
                             lambda j, idx_ref: (idx_ref[0], j, 0)),
                pl.BlockSpec((1, _TILE, d),
                             lambda j, idx_ref: (idx_ref[1], j, 0)),
                pl.BlockSpec((1, n_tiles, _TILE),
                             lambda j, idx_ref: (idx_ref[0], 0, 0)),
                pl.BlockSpec((1, n_tiles, _TILE),
                             lambda j, idx_ref: (idx_ref[1], 0, 0)),
                pl.BlockSpec((T, 2), lambda j, idx_ref: (0, 0)),
            ],
            out_specs=pl.BlockSpec((T, _TILE), lambda j, idx_ref: (0, j)),
        ),
        out_shape=jax.ShapeDtypeStruct((T, d), jnp.float32),
    )(idx, x, expert_w, expert_w, b3, b3, rw)

    return out.reshape(B, S, d)


# speedup vs baseline: 1.0067x; 1.0067x over previous
import jax
import jax.numpy as jnp
from jax.experimental import pallas as pl
from jax.experimental.pallas import tpu as pltpu

_TILE = 512
_HALF = 1024


def _gate_kernel(x_ref, gw_ref, idx_ref, rw_ref):
    x = x_ref[...]
    logits = jax.lax.dot_general(
        x, gw_ref[...], (((1,), (1,)), ((), ())),
        preferred_element_type=jnp.float32)
    s = jnp.sum(logits, axis=0, keepdims=True)
    e_iota = jax.lax.broadcasted_iota(jnp.int32, s.shape, 1)
    i0 = jnp.argmax(s, axis=1)[0]
    s_masked = jnp.where(e_iota == i0, -jnp.inf, s)
    i1 = jnp.argmax(s_masked, axis=1)[0]

    l0 = jnp.sum(jnp.where(e_iota == i0, logits, 0.0), axis=1, keepdims=True)
    l1 = jnp.sum(jnp.where(e_iota == i1, logits, 0.0), axis=1, keepdims=True)
    m = jnp.maximum(l0, l1)
    e0 = jnp.exp(l0 - m)
    e1 = jnp.exp(l1 - m)
    denom = e0 + e1
    w0 = e0 / denom
    w1 = e1 / denom

    k_iota = jax.lax.broadcasted_iota(jnp.int32, (1, 2), 1)
    idx_ref[...] = jnp.where(k_iota == 0, i0, i1).astype(jnp.int32)
    rw_ref[...] = jnp.concatenate([w0, w1], axis=1)


def _expert_kernel(idx_ref, x_ref, w0a_ref, w0b_ref, w1a_ref, w1b_ref,
                   b0_ref, b1_ref, rw_ref, out_ref):
    x = x_ref[...]
    xa = x[:, :_HALF]
    xb = x[:, _HALF:]
    dn = (((1,), (1,)), ((), ()))
    part0 = (jax.lax.dot_general(xa, w0a_ref[0], dn,
                                 preferred_element_type=jnp.float32) +
             jax.lax.dot_general(xb, w0b_ref[0], dn,
                                 preferred_element_type=jnp.float32))
    part1 = (jax.lax.dot_general(xa, w1a_ref[0], dn,
                                 preferred_element_type=jnp.float32) +
             jax.lax.dot_general(xb, w1b_ref[0], dn,
                                 preferred_element_type=jnp.float32))
    rw = rw_ref[...]
    w0 = rw[:, 0:1]
    w1 = rw[:, 1:2]
    j = pl.program_id(0)
    b0 = b0_ref[0, j, :].reshape(1, -1)
    b1 = b1_ref[0, j, :].reshape(1, -1)
    out_ref[...] = w0 * (part0 + b0) + w1 * (part1 + b1)


@jax.jit
def kernel(hidden_states, gate_w, expert_w, expert_b):
    B, S, d = hidden_states.shape
    T = B * S
    x = hidden_states.reshape(T, d)

    idx2d, rw = pl.pallas_call(
        _gate_kernel,
        out_shape=(
            jax.ShapeDtypeStruct((1, 2), jnp.int32),
            jax.ShapeDtypeStruct((T, 2), jnp.float32),
        ),
    )(x, gate_w)
    idx = idx2d.reshape(2)

    n_tiles = d // _TILE
    b3 = expert_b.reshape(expert_b.shape[0], n_tiles, _TILE)
    wspec_a = lambda k: pl.BlockSpec(
        (1, _TILE, _HALF), lambda j, idx_ref, k=k: (idx_ref[k], j, 0))
    wspec_b = lambda k: pl.BlockSpec(
        (1, _TILE, _HALF), lambda j, idx_ref, k=k: (idx_ref[k], j, 1))
    out = pl.pallas_call(
        _expert_kernel,
        grid_spec=pltpu.PrefetchScalarGridSpec(
            num_scalar_prefetch=1,
            grid=(n_tiles,),
            in_specs=[
                pl.BlockSpec((T, d), lambda j, idx_ref: (0, 0)),
                wspec_a(0),
                wspec_b(0),
                wspec_a(1),
                wspec_b(1),
                pl.BlockSpec((1, n_tiles, _TILE),
                             lambda j, idx_ref: (idx_ref[0], 0, 0)),
                pl.BlockSpec((1, n_tiles, _TILE),
                             lambda j, idx_ref: (idx_ref[1], 0, 0)),
                pl.BlockSpec((T, 2), lambda j, idx_ref: (0, 0)),
            ],
            out_specs=pl.BlockSpec((T, _TILE), lambda j, idx_ref: (0, j)),
        ),
        out_shape=jax.ShapeDtypeStruct((T, d), jnp.float32),
    )(idx, x, expert_w, expert_w, expert_w, expert_w, b3, b3, rw)

    return out.reshape(B, S, d)
